# Initial kernel scaffold; baseline (speedup 1.0000x reference)
import jax
import jax.numpy as jnp
from jax.experimental import pallas as pl


def kernel(query_length, key_length, W):
    raise NotImplementedError("write your pallas kernel here")


# speedup vs baseline: 36.9363x; 36.9363x over previous
import functools

import jax
import jax.numpy as jnp
from jax import lax
from jax.experimental import pallas as pl
from jax.experimental.pallas import tpu as pltpu
from jax.experimental.pallas import tpu_sc as plsc

_MAXD = 128
_H = 16
_Q = 2048
_K = 2048
_EXT = 4096
_ROWS_PER_WORKER = _Q // 2
_BANDP = 272
_BAND_LO = _Q - 1 - _MAXD


def _rpb_body(wt_hbm, out_hbm, band_v, e_v):
    wid = lax.axis_index("s") * 2 + lax.axis_index("c")
    h = wid >> 1
    qbase = (wid & 1) * _ROWS_PER_WORKER

    pltpu.sync_copy(
        wt_hbm.at[pl.ds(pl.multiple_of(h * _BANDP, 8), _BANDP)], band_v
    )

    zeros = jnp.zeros((16,), jnp.float32)
    v_lo = zeros + band_v[pl.ds(0, 16)][0]
    v_hi = zeros + band_v[pl.ds(2 * _MAXD, 16)][0]

    for r in range(8):
        roff = r * _EXT

        def fill_lo(c, carry, roff=roff):
            e_v[pl.ds(roff + c * 16, 16)] = v_lo
            return carry

        def fill_hi(c, carry, roff=roff):
            e_v[pl.ds(roff + 2176 + c * 16, 16)] = v_hi
            return carry

        lax.fori_loop(0, 1920 // 16, fill_lo, 0)
        lax.fori_loop(0, (_EXT - 2176) // 16, fill_hi, 0)
        for c in range(_BANDP // 16):
            e_v[pl.ds(roff + _BAND_LO - r + c * 16, 16)] = band_v[
                pl.ds(c * 16, 16)
            ]

    for r in range(8):
        def emit(j, carry, r=r):
            q = qbase + (7 - r) + 8 * j
            base = pl.multiple_of((_Q - 1) - q - r, 8)
            dst = pl.multiple_of((h * _Q + q) * _K, 8)
            pltpu.sync_copy(
                e_v.at[pl.ds(r * _EXT + base, _K)], out_hbm.at[pl.ds(dst, _K)]
            )
            return carry

        lax.fori_loop(0, _ROWS_PER_WORKER // 8, emit, 0)


def kernel(query_length, key_length, W):
    wt = jnp.concatenate(
        [W.T, jnp.broadcast_to(W.T[:, -1:], (_H, _BANDP - (2 * _MAXD + 1)))],
        axis=1,
    ).reshape(-1)
    mesh = plsc.VectorSubcoreMesh(core_axis_name="c", subcore_axis_name="s")
    run = functools.partial(
        pl.kernel,
        mesh=mesh,
        out_type=jax.ShapeDtypeStruct((_H, _Q, _K), jnp.float32),
        scratch_types=[
            pltpu.VMEM((_BANDP,), jnp.float32),
            pltpu.VMEM((8 * _EXT,), jnp.float32),
        ],
    )(_rpb_body)
    out = run(wt)
    return out.reshape(1, _H, _Q, _K)


# speedup vs baseline: 42.4997x; 1.1506x over previous
import functools

import jax
import jax.numpy as jnp
from jax import lax
from jax.experimental import pallas as pl
from jax.experimental.pallas import tpu as pltpu
from jax.experimental.pallas import tpu_sc as plsc

_MAXD = 128
_H = 16
_Q = 2048
_K = 2048
_EXT = 4096
_ROWS_PER_WORKER = _Q // 2
_BANDP = 272
_BAND_LO = _Q - 1 - _MAXD


def _rpb_body(wt_hbm, out_hbm, band_v, e_v, sem):
    wid = lax.axis_index("s") * 2 + lax.axis_index("c")
    h = wid >> 1
    qbase = (wid & 1) * _ROWS_PER_WORKER

    pltpu.sync_copy(
        wt_hbm.at[pl.ds(pl.multiple_of(h * _BANDP, 8), _BANDP)], band_v
    )

    zeros = jnp.zeros((16,), jnp.float32)
    v_lo = zeros + band_v[pl.ds(0, 16)][0]
    v_hi = zeros + band_v[pl.ds(2 * _MAXD, 16)][0]

    for r in range(8):
        roff = r * _EXT

        def fill_lo(c, carry, roff=roff):
            e_v[pl.ds(roff + c * 16, 16)] = v_lo
            return carry

        def fill_hi(c, carry, roff=roff):
            e_v[pl.ds(roff + 2176 + c * 16, 16)] = v_hi
            return carry

        lax.fori_loop(0, 1920 // 16, fill_lo, 0)
        lax.fori_loop(0, (_EXT - 2176) // 16, fill_hi, 0)
        for c in range(_BANDP // 16):
            e_v[pl.ds(roff + _BAND_LO - r + c * 16, 16)] = band_v[
                pl.ds(c * 16, 16)
            ]

    depth = 8

    def emit(i, carry):
        q = qbase + i
        start = (_Q - 1) - q
        r = start & 7
        src = pl.multiple_of(r * _EXT + (start - r), 8)
        dst = pl.multiple_of((h * _Q + q) * _K, 8)
        pltpu.async_copy(
            e_v.at[pl.ds(src, _K)], out_hbm.at[pl.ds(dst, _K)], sem
        )

        @pl.when(i >= depth)
        def _wait_one():
            pltpu.make_async_copy(
                e_v.at[pl.ds(0, _K)], out_hbm.at[pl.ds(0, _K)], sem
            ).wait()

        return carry

    lax.fori_loop(0, _ROWS_PER_WORKER, emit, 0)
    for _ in range(depth):
        pltpu.make_async_copy(
            e_v.at[pl.ds(0, _K)], out_hbm.at[pl.ds(0, _K)], sem
        ).wait()


def kernel(query_length, key_length, W):
    wt = jnp.concatenate(
        [W.T, jnp.broadcast_to(W.T[:, -1:], (_H, _BANDP - (2 * _MAXD + 1)))],
        axis=1,
    ).reshape(-1)
    mesh = plsc.VectorSubcoreMesh(core_axis_name="c", subcore_axis_name="s")
    run = functools.partial(
        pl.kernel,
        mesh=mesh,
        out_type=jax.ShapeDtypeStruct((_H, _Q, _K), jnp.float32),
        scratch_types=[
            pltpu.VMEM((_BANDP,), jnp.float32),
            pltpu.VMEM((8 * _EXT,), jnp.float32),
            pltpu.SemaphoreType.DMA,
        ],
    )(_rpb_body)
    out = run(wt)
    return out.reshape(1, _H, _Q, _K)
